# Initial kernel scaffold; baseline (speedup 1.0000x reference)
import jax
import jax.numpy as jnp
from jax.experimental import pallas as pl


def kernel(x, table):
    raise NotImplementedError("write your pallas kernel here")


# speedup vs baseline: 1.4930x; 1.4930x over previous
import functools

import jax
import jax.numpy as jnp
from jax import lax
from jax.experimental import pallas as pl
from jax.experimental.pallas import tpu as pltpu
from jax.experimental.pallas import tpu_sc as plsc

EMBED_DIM = 32
BATCH = 4096
HIST = 200
TOTAL = BATCH * HIST
NUM_CORES = 2
NUM_SUBCORES = 16
NW = NUM_CORES * NUM_SUBCORES
PER_W = TOTAL // NW
CHUNK = 128
NCHUNK = PER_W // CHUNK
NBUF = 4

_mesh = plsc.VectorSubcoreMesh(core_axis_name="c", subcore_axis_name="s")


@functools.partial(
    pl.kernel,
    mesh=_mesh,
    out_type=jax.ShapeDtypeStruct((TOTAL, EMBED_DIM), jnp.float32),
    scratch_types=[
        pltpu.VMEM((NCHUNK, CHUNK), jnp.int32),
        pltpu.VMEM((NBUF, CHUNK, EMBED_DIM), jnp.float32),
        pltpu.SemaphoreType.DMA,
        pltpu.SemaphoreType.DMA,
        pltpu.SemaphoreType.DMA,
        pltpu.SemaphoreType.DMA,
    ],
    compiler_params=pltpu.CompilerParams(use_tc_tiling_on_sc=False),
)
def _emb_gather(x_hbm, table_hbm, out_hbm, idx_v, rows_v, s0, s1, s2, s3):
    sems = (s0, s1, s2, s3)
    w = lax.axis_index("s") * NUM_CORES + lax.axis_index("c")
    base = w * PER_W

    pltpu.sync_copy(x_hbm.at[w], idx_v)

    for b in range(NBUF):
        pltpu.async_copy(table_hbm.at[idx_v.at[b]], rows_v.at[b], sems[b])

    def body(i, _):
        g0 = i * NBUF
        for b in range(NBUF):
            g = g0 + b
            pltpu.make_async_copy(
                table_hbm.at[idx_v.at[g]], rows_v.at[b], sems[b]
            ).wait()
            pltpu.sync_copy(rows_v.at[b], out_hbm.at[pl.ds(base + g * CHUNK, CHUNK)])
            nxt = g + NBUF

            @pl.when(nxt < NCHUNK)
            def _start_next():
                pltpu.async_copy(
                    table_hbm.at[idx_v.at[nxt]], rows_v.at[b], sems[b]
                )

        return _

    lax.fori_loop(0, NCHUNK // NBUF, body, None)


@jax.jit
def kernel(x, table):
    xf = x.astype(jnp.int32).reshape(NW, NCHUNK, CHUNK)
    out = _emb_gather(xf, table)
    return out.reshape(BATCH, HIST, EMBED_DIM)


# speedup vs baseline: 1.4988x; 1.0039x over previous
import functools

import jax
import jax.numpy as jnp
from jax import lax
from jax.experimental import pallas as pl
from jax.experimental.pallas import tpu as pltpu
from jax.experimental.pallas import tpu_sc as plsc

EMBED_DIM = 32
BATCH = 4096
HIST = 200
TOTAL = BATCH * HIST
NUM_CORES = 2
NUM_SUBCORES = 16
NW = NUM_CORES * NUM_SUBCORES
PER_W = TOTAL // NW
CHUNK = 128
NCHUNK = PER_W // CHUNK
NBUF = 8

_mesh = plsc.VectorSubcoreMesh(core_axis_name="c", subcore_axis_name="s")


@functools.partial(
    pl.kernel,
    mesh=_mesh,
    out_type=jax.ShapeDtypeStruct((TOTAL, EMBED_DIM), jnp.float32),
    scratch_types=[
        pltpu.VMEM((NCHUNK, CHUNK), jnp.int32),
        pltpu.VMEM((NBUF, CHUNK, EMBED_DIM), jnp.float32),
        [pltpu.SemaphoreType.DMA] * NBUF,
        [pltpu.SemaphoreType.DMA] * NBUF,
    ],
    compiler_params=pltpu.CompilerParams(use_tc_tiling_on_sc=False),
)
def _emb_gather(x_hbm, table_hbm, out_hbm, idx_v, rows_v, gsems, wsems):
    w = lax.axis_index("s") * NUM_CORES + lax.axis_index("c")
    base = w * PER_W

    pltpu.sync_copy(x_hbm.at[w], idx_v)

    for b in range(NBUF):
        pltpu.async_copy(table_hbm.at[idx_v.at[b]], rows_v.at[b], gsems[b])

    def body(i, _):
        g0 = i * NBUF
        for b in range(NBUF):
            g = g0 + b
            pltpu.make_async_copy(
                table_hbm.at[idx_v.at[g]], rows_v.at[b], gsems[b]
            ).wait()
            pltpu.async_copy(
                rows_v.at[b], out_hbm.at[pl.ds(base + g * CHUNK, CHUNK)], wsems[b]
            )
        for b in range(NBUF):
            g = g0 + b
            nxt = g + NBUF

            @pl.when(nxt < NCHUNK)
            def _reuse():
                pltpu.make_async_copy(
                    rows_v.at[b], out_hbm.at[pl.ds(base + g * CHUNK, CHUNK)], wsems[b]
                ).wait()
                pltpu.async_copy(
                    table_hbm.at[idx_v.at[nxt]], rows_v.at[b], gsems[b]
                )

        return _

    lax.fori_loop(0, NCHUNK // NBUF, body, None)

    for b in range(NBUF):
        g = NCHUNK - NBUF + b
        pltpu.make_async_copy(
            rows_v.at[b], out_hbm.at[pl.ds(base + g * CHUNK, CHUNK)], wsems[b]
        ).wait()


@jax.jit
def kernel(x, table):
    xf = x.astype(jnp.int32).reshape(NW, NCHUNK, CHUNK)
    out = _emb_gather(xf, table)
    return out.reshape(BATCH, HIST, EMBED_DIM)
